# Initial kernel scaffold; baseline (speedup 1.0000x reference)
import jax
import jax.numpy as jnp
from jax.experimental import pallas as pl


def kernel(x, pos, W_local, b_local, W_gate, b_gate, edge_index):
    raise NotImplementedError("write your pallas kernel here")


# speedup vs baseline: 6.0821x; 6.0821x over previous
import functools

import jax
import jax.numpy as jnp
from jax import lax
from jax.experimental import pallas as pl
from jax.experimental.pallas import tpu as pltpu
from jax.experimental.pallas import tpu_sc as plsc

NC, NS, L = 2, 16, 16
NW = NC * NS
C = 128
D = 128
BE = 1024


def _prep_body(x_ref, p_ref, w1_ref, w2_ref, b_ref, a_ref, bb_ref):
    pv = p_ref[...] @ w2_ref[...]
    a_ref[...] = x_ref[...] @ w1_ref[...] + pv + b_ref[...]
    bb_ref[...] = pv


def _dense_body(as_ref, bd_ref, wg_ref, bg_ref, msg_ref, gate_ref):
    z = as_ref[...] - bd_ref[...]
    m = z * jax.nn.sigmoid(z)
    msg_ref[...] = m
    g = jnp.sum(m * wg_ref[...], axis=1, keepdims=True) + bg_ref[...]
    gate_ref[...] = g * jax.nn.sigmoid(g)


def _comb_body(p0_ref, p1_ref, o_ref):
    o_ref[...] = p0_ref[...] + p1_ref[...]


def _make_gather(Epad, Nacc):
    mesh = plsc.VectorSubcoreMesh(core_axis_name="c", subcore_axis_name="s",
                                  num_cores=NC, num_subcores=NS)
    f32 = jnp.float32

    @functools.partial(
        pl.kernel, mesh=mesh,
        out_type=(jax.ShapeDtypeStruct((Epad, D), f32),
                  jax.ShapeDtypeStruct((Epad, D), f32)),
        scratch_types=[
            pltpu.VMEM((C,), jnp.int32), pltpu.VMEM((C,), jnp.int32),
            pltpu.VMEM((C, D), f32), pltpu.VMEM((C, D), f32),
            pltpu.SemaphoreType.DMA, pltpu.SemaphoreType.DMA,
        ],
    )
    def k(a_hbm, b_hbm, src_hbm, dst_hbm, as_hbm, bd_hbm,
          sidx, didx, arows, brows, sem1, sem2):
        wid = lax.axis_index("s") * NC + lax.axis_index("c")
        ept = Epad // NW
        base0 = wid * ept

        def body(i, carry):
            base = base0 + i * C
            pltpu.sync_copy(src_hbm.at[pl.ds(base, C)], sidx)
            pltpu.sync_copy(dst_hbm.at[pl.ds(base, C)], didx)
            ca = pltpu.async_copy(a_hbm.at[sidx], arows, sem1)
            cb = pltpu.async_copy(b_hbm.at[didx], brows, sem2)
            ca.wait()
            cb.wait()
            pltpu.sync_copy(arows, as_hbm.at[pl.ds(base, C)])
            pltpu.sync_copy(brows, bd_hbm.at[pl.ds(base, C)])
            return carry

        lax.fori_loop(0, ept // C, body, 0)

    return k


def _make_agg(Epad, Nacc):
    mesh = plsc.VectorSubcoreMesh(core_axis_name="c", subcore_axis_name="s",
                                  num_cores=NC, num_subcores=NS)
    f32 = jnp.float32
    SEG = Nacc // NS

    @functools.partial(
        pl.kernel, mesh=mesh,
        out_type=(jax.ShapeDtypeStruct((Nacc, D), f32),
                  jax.ShapeDtypeStruct((Nacc, D), f32)),
        scratch_types=[
            pltpu.VMEM((C,), jnp.int32),
            pltpu.VMEM((C,), f32),
            pltpu.VMEM((C,), f32),
            pltpu.VMEM((C,), f32),
            pltpu.VMEM((C, D), f32),
            pltpu.VMEM((1, L), f32),
            pltpu.VMEM((NS, L), f32),
            pltpu.VMEM_SHARED((Nacc,), f32),
            pltpu.VMEM_SHARED((Nacc, D), f32),
            pltpu.VMEM_SHARED((NS, L), f32),
            pltpu.SemaphoreType.DMA,
        ],
    )
    def k(gate_hbm, dst_hbm, msg_hbm, zrow_hbm, zacc_hbm, p0_hbm, p1_hbm,
          didx, gbuf, ebuf, dbuf, mrows, mx_v, mall_v,
          denom_sh, acc_sh, maxima_sh, sem):
        cid = lax.axis_index("c")
        sid = lax.axis_index("s")
        wid = sid * NC + cid
        ept16 = Epad // NS
        eptw = Epad // NW

        pltpu.sync_copy(zrow_hbm, denom_sh.at[pl.ds(sid * SEG, SEG)])
        pltpu.sync_copy(zacc_hbm, acc_sh.at[pl.ds(sid * SEG, SEG)])

        neg = jnp.full((L,), -1e30, f32)

        def amax_body(i, m):
            pltpu.sync_copy(gate_hbm.at[pl.ds(sid * ept16 + i * C, C)], gbuf)
            for j in range(C // L):
                m = jnp.maximum(m, gbuf[pl.ds(j * L, L)])
            return m

        m = lax.fori_loop(0, ept16 // C, amax_body, neg)
        mx_v[0, :] = m
        pltpu.sync_copy(mx_v, maxima_sh.at[pl.ds(sid, 1)])
        plsc.subcore_barrier()
        pltpu.sync_copy(maxima_sh, mall_v)
        gm = neg
        for s in range(NS):
            gm = jnp.maximum(gm, mall_v[s])
        lane = lax.iota(jnp.int32, L)
        for sh in (1, 2, 4, 8):
            gm = jnp.maximum(gm, gm[lane ^ sh])
        G = gm

        def db(i, carry):
            base = sid * ept16 + i * C
            pltpu.sync_copy(gate_hbm.at[pl.ds(base, C)], gbuf)
            pltpu.sync_copy(dst_hbm.at[pl.ds(base, C)], didx)
            for j in range(C // L):
                ebuf[pl.ds(j * L, L)] = jnp.exp(gbuf[pl.ds(j * L, L)] - G)
            pltpu.sync_copy(ebuf, denom_sh.at[didx], add=True)
            return carry

        lax.fori_loop(0, ept16 // C, db, 0)
        plsc.subcore_barrier()

        def wb(i, carry):
            base = wid * eptw + i * C
            pltpu.sync_copy(gate_hbm.at[pl.ds(base, C)], gbuf)
            pltpu.sync_copy(dst_hbm.at[pl.ds(base, C)], didx)
            pltpu.async_copy(msg_hbm.at[pl.ds(base, C)], mrows, sem).wait()
            pltpu.async_copy(denom_sh.at[didx], dbuf, sem).wait()
            for j in range(C // L):
                a = jnp.exp(gbuf[pl.ds(j * L, L)] - G) / (
                    dbuf[pl.ds(j * L, L)] + 1e-16)
                ebuf[pl.ds(j * L, L)] = a

            def rowb(g, carry2):
                av = ebuf[pl.ds(g * L, L)]
                for l in range(L):
                    bv = jnp.full((L,), av[l], f32)
                    e = g * L + l
                    for j in range(D // L):
                        mrows[e, pl.ds(j * L, L)] = (
                            mrows[e, pl.ds(j * L, L)] * bv)
                return carry2

            lax.fori_loop(0, C // L, rowb, 0)
            pltpu.sync_copy(mrows, acc_sh.at[didx], add=True)
            return carry

        lax.fori_loop(0, eptw // C, wb, 0)
        plsc.subcore_barrier()

        @pl.when(cid == 0)
        def _():
            pltpu.sync_copy(acc_sh.at[pl.ds(sid * SEG, SEG)],
                            p0_hbm.at[pl.ds(sid * SEG, SEG)])

        @pl.when(cid == 1)
        def _():
            pltpu.sync_copy(acc_sh.at[pl.ds(sid * SEG, SEG)],
                            p1_hbm.at[pl.ds(sid * SEG, SEG)])

    return k


def kernel(x, pos, W_local, b_local, W_gate, b_gate, edge_index):
    f32 = jnp.float32
    N = x.shape[0]
    E = edge_index.shape[1]

    loops = jnp.arange(N, dtype=edge_index.dtype)
    src = jnp.concatenate([edge_index[0], loops])
    dst = jnp.concatenate([edge_index[1], loops])
    Et = E + N
    Epad = ((Et + NW * C - 1) // (NW * C)) * (NW * C)
    Nacc = ((N + NS * 8 - 1) // (NS * 8)) * (NS * 8) + NS * 8
    pad_idx = N + 4
    pad = jnp.full((Epad - Et,), pad_idx, dtype=src.dtype)
    src = jnp.concatenate([src, pad])
    dst = jnp.concatenate([dst, pad])

    xp = jnp.zeros((Nacc, D), f32).at[:N].set(x)
    posP = jnp.zeros((Nacc, D), f32).at[:N, :3].set(pos)
    W1 = W_local[:D]
    W2 = jnp.zeros((D, D), f32).at[:3].set(W_local[D:])

    A, B = pl.pallas_call(
        _prep_body,
        out_shape=(jax.ShapeDtypeStruct((Nacc, D), f32),
                   jax.ShapeDtypeStruct((Nacc, D), f32)),
    )(xp, posP, W1, W2, b_local.reshape(1, D))

    AS, BD = _make_gather(Epad, Nacc)(A, B, src, dst)

    nblk = Epad // BE
    msg, gcol = pl.pallas_call(
        _dense_body,
        grid=(nblk,),
        in_specs=[
            pl.BlockSpec((BE, D), lambda i: (i, 0)),
            pl.BlockSpec((BE, D), lambda i: (i, 0)),
            pl.BlockSpec((1, D), lambda i: (0, 0)),
            pl.BlockSpec((1, 1), lambda i: (0, 0)),
        ],
        out_specs=[
            pl.BlockSpec((BE, D), lambda i: (i, 0)),
            pl.BlockSpec((BE, 1), lambda i: (i, 0)),
        ],
        out_shape=(jax.ShapeDtypeStruct((Epad, D), f32),
                   jax.ShapeDtypeStruct((Epad, 1), f32)),
    )(AS, BD, W_gate.reshape(1, D), b_gate.reshape(1, 1))
    gate = gcol.reshape(Epad)

    SEG = Nacc // NS
    zrow = jnp.zeros((SEG,), f32)
    zacc = jnp.zeros((SEG, D), f32)
    P0, P1 = _make_agg(Epad, Nacc)(gate, dst, msg, zrow, zacc)

    NB = 2000
    out = pl.pallas_call(
        _comb_body,
        grid=(N // NB,),
        in_specs=[pl.BlockSpec((NB, D), lambda i: (i, 0)),
                  pl.BlockSpec((NB, D), lambda i: (i, 0))],
        out_specs=pl.BlockSpec((NB, D), lambda i: (i, 0)),
        out_shape=jax.ShapeDtypeStruct((N, D), f32),
    )(P0, P1)
    return out
